# Initial kernel scaffold; baseline (speedup 1.0000x reference)
import jax
import jax.numpy as jnp
from jax.experimental import pallas as pl


def kernel(x, edge_index, W1, b1, W2, b2, W3, b3, M1, mb1, M2, mb2, M3, mb3):
    raise NotImplementedError("write your pallas kernel here")


# speedup vs baseline: 16.9059x; 16.9059x over previous
import functools

import jax
import jax.numpy as jnp
from jax import lax
from jax.experimental import pallas as pl
from jax.experimental.pallas import tpu as pltpu
from jax.experimental.pallas import tpu_sc as plsc

N = 10000
E = 320000
NPAD = 10240
NC = 2
NS = 16
NW = NC * NS
CHUNK = 128
EPT = 10240
E_PAD = EPT * NW
NB = 4
ROWS_PER_TILE = NPAD // NS
BLK = 1024
GRID = NPAD // BLK


def _sc_mesh():
    return plsc.VectorSubcoreMesh(core_axis_name="c", subcore_axis_name="s")


_SC_PARAMS = pltpu.CompilerParams(use_tc_tiling_on_sc=False)


def _deg_kernel(dst_pad, ones_hbm, zeros_hbm):
    @functools.partial(
        pl.kernel,
        out_type=jax.ShapeDtypeStruct((NC, NPAD, 16), jnp.float32),
        mesh=_sc_mesh(),
        scratch_types=[
            pltpu.VMEM((CHUNK, 16), jnp.float32),
            pltpu.VMEM((1, CHUNK), jnp.int32),
            pltpu.VMEM_SHARED((NPAD, 16), jnp.float32),
        ],
        compiler_params=_SC_PARAMS,
    )
    def k(dst_hbm, ones_h, zeros_h, out_hbm, ones_v, idx_v, acc):
        c = lax.axis_index("c")
        s = lax.axis_index("s")
        wid = c * NS + s
        pltpu.sync_copy(ones_h, ones_v)
        pltpu.sync_copy(zeros_h, acc.at[pl.ds(s * ROWS_PER_TILE, ROWS_PER_TILE)])
        plsc.subcore_barrier()

        ebase = wid * EPT

        def body(i, _):
            pltpu.sync_copy(dst_hbm.at[pl.ds(ebase + i * CHUNK, CHUNK)],
                            idx_v.at[0])
            pltpu.sync_copy(ones_v, acc.at[idx_v.at[0]], add=True)
            return 0

        lax.fori_loop(0, EPT // CHUNK, body, 0)
        plsc.subcore_barrier()
        pltpu.sync_copy(acc.at[pl.ds(s * ROWS_PER_TILE, ROWS_PER_TILE)],
                        out_hbm.at[c, pl.ds(s * ROWS_PER_TILE, ROWS_PER_TILE)])

    return k(dst_pad, ones_hbm, zeros_hbm)


def _agg_kernel(F, hs, src_pad, dst_pad, zeros_hbm):
    @functools.partial(
        pl.kernel,
        out_type=jax.ShapeDtypeStruct((NC, NPAD, F), jnp.float32),
        mesh=_sc_mesh(),
        scratch_types=[
            pltpu.VMEM((NB, CHUNK), jnp.int32),
            pltpu.VMEM((NB, CHUNK), jnp.int32),
            pltpu.VMEM((NB, CHUNK, F), jnp.float32),
            pltpu.VMEM_SHARED((NPAD, F), jnp.float32),
            tuple(pltpu.SemaphoreType.DMA for _ in range(NB)),
        ],
        compiler_params=_SC_PARAMS,
    )
    def k(hs_hbm, src_hbm, dst_hbm, zeros_h, out_hbm, sidx, didx, rows, acc, sems):
        c = lax.axis_index("c")
        s = lax.axis_index("s")
        wid = c * NS + s
        pltpu.sync_copy(zeros_h, acc.at[pl.ds(s * ROWS_PER_TILE, ROWS_PER_TILE)])
        plsc.subcore_barrier()

        ebase = wid * EPT

        def body(it, _):
            base = ebase + it * (NB * CHUNK)
            descs = []
            for b in range(NB):
                pltpu.sync_copy(src_hbm.at[pl.ds(base + b * CHUNK, CHUNK)],
                                sidx.at[b])
                pltpu.sync_copy(dst_hbm.at[pl.ds(base + b * CHUNK, CHUNK)],
                                didx.at[b])
                descs.append(
                    pltpu.async_copy(hs_hbm.at[sidx.at[b]], rows.at[b], sems[b]))
            for b in range(NB):
                descs[b].wait()
                pltpu.sync_copy(rows.at[b], acc.at[didx.at[b]], add=True)
            return 0

        lax.fori_loop(0, EPT // (NB * CHUNK), body, 0)
        plsc.subcore_barrier()
        pltpu.sync_copy(acc.at[pl.ds(s * ROWS_PER_TILE, ROWS_PER_TILE)],
                        out_hbm.at[c, pl.ds(s * ROWS_PER_TILE, ROWS_PER_TILE)])

    return k(hs, src_pad, dst_pad, zeros_hbm)


def _tc_first(xpad, W1, degP):
    def body(x_ref, w_ref, d_ref, hs_ref, dinv_ref):
        deg = d_ref[0] + d_ref[1] + 1.0
        dinv = lax.rsqrt(deg)
        dinv_ref[...] = dinv
        h = jnp.dot(x_ref[...], w_ref[...],
                    preferred_element_type=jnp.float32)
        hs_ref[...] = h * dinv

    return pl.pallas_call(
        body,
        grid=(GRID,),
        in_specs=[
            pl.BlockSpec((BLK, 128), lambda i: (i, 0)),
            pl.BlockSpec((128, 16), lambda i: (0, 0)),
            pl.BlockSpec((2, BLK, 16), lambda i: (0, i, 0)),
        ],
        out_specs=[
            pl.BlockSpec((BLK, 16), lambda i: (i, 0)),
            pl.BlockSpec((BLK, 16), lambda i: (i, 0)),
        ],
        out_shape=[
            jax.ShapeDtypeStruct((NPAD, 16), jnp.float32),
            jax.ShapeDtypeStruct((NPAD, 16), jnp.float32),
        ],
    )(xpad, W1, degP)


def _tc_mid(F, Fn, aggP, hs, dinv16, b, Wn):
    def body(a_ref, hs_ref, dinv_ref, b_ref, w_ref, out_ref):
        dinv_f = jnp.concatenate([dinv_ref[...]] * (F // 16), axis=1)
        agg = a_ref[0] + a_ref[1] + hs_ref[...]
        h = jax.nn.relu(agg * dinv_f + b_ref[...])
        hn = jnp.dot(h, w_ref[...], preferred_element_type=jnp.float32)
        dinv_fn = jnp.concatenate([dinv_ref[...]] * (Fn // 16), axis=1)
        out_ref[...] = hn * dinv_fn

    return pl.pallas_call(
        body,
        grid=(GRID,),
        in_specs=[
            pl.BlockSpec((2, BLK, F), lambda i: (0, i, 0)),
            pl.BlockSpec((BLK, F), lambda i: (i, 0)),
            pl.BlockSpec((BLK, 16), lambda i: (i, 0)),
            pl.BlockSpec((1, F), lambda i: (0, 0)),
            pl.BlockSpec((F, Fn), lambda i: (0, 0)),
        ],
        out_specs=pl.BlockSpec((BLK, Fn), lambda i: (i, 0)),
        out_shape=jax.ShapeDtypeStruct((NPAD, Fn), jnp.float32),
    )(aggP, hs, dinv16, b, Wn)


def _tc_final(aggP, hs, dinv16, b3, M1, mb1, M2, mb2, M3, mb3):
    def body(a_ref, hs_ref, dinv_ref, b3_ref, m1_ref, mb1_ref, m2_ref,
             mb2_ref, m3_ref, mb3_ref, out_ref):
        dinv64 = jnp.concatenate([dinv_ref[...]] * 4, axis=1)
        agg = a_ref[0] + a_ref[1] + hs_ref[...]
        h = jax.nn.relu(agg * dinv64 + b3_ref[...])
        h = jax.nn.relu(jnp.dot(h, m1_ref[...],
                                preferred_element_type=jnp.float32)
                        + mb1_ref[...])
        h = jax.nn.relu(jnp.dot(h, m2_ref[...],
                                preferred_element_type=jnp.float32)
                        + mb2_ref[...])
        out_ref[...] = (jnp.dot(h, m3_ref[...],
                                preferred_element_type=jnp.float32)
                        + mb3_ref[...])

    return pl.pallas_call(
        body,
        grid=(GRID,),
        in_specs=[
            pl.BlockSpec((2, BLK, 64), lambda i: (0, i, 0)),
            pl.BlockSpec((BLK, 64), lambda i: (i, 0)),
            pl.BlockSpec((BLK, 16), lambda i: (i, 0)),
            pl.BlockSpec((1, 64), lambda i: (0, 0)),
            pl.BlockSpec((64, 32), lambda i: (0, 0)),
            pl.BlockSpec((1, 32), lambda i: (0, 0)),
            pl.BlockSpec((32, 16), lambda i: (0, 0)),
            pl.BlockSpec((1, 16), lambda i: (0, 0)),
            pl.BlockSpec((16, 1), lambda i: (0, 0)),
            pl.BlockSpec((1, 1), lambda i: (0, 0)),
        ],
        out_specs=pl.BlockSpec((BLK, 1), lambda i: (i, 0)),
        out_shape=jax.ShapeDtypeStruct((NPAD, 1), jnp.float32),
    )(aggP, hs, dinv16, b3, M1, mb1, M2, mb2, M3, mb3)


def kernel(x, edge_index, W1, b1, W2, b2, W3, b3, M1, mb1, M2, mb2, M3, mb3):
    src = edge_index[0]
    dst = edge_index[1]
    src_pad = jnp.concatenate(
        [src, jnp.zeros((E_PAD - E,), jnp.int32)])
    dst_pad = jnp.concatenate(
        [dst, jnp.full((E_PAD - E,), N, jnp.int32)])
    xpad = jnp.pad(x, ((0, NPAD - N), (0, 0)))

    ones16 = jnp.ones((CHUNK, 16), jnp.float32)
    z16 = jnp.zeros((ROWS_PER_TILE, 16), jnp.float32)
    z32 = jnp.zeros((ROWS_PER_TILE, 32), jnp.float32)
    z64 = jnp.zeros((ROWS_PER_TILE, 64), jnp.float32)

    degP = _deg_kernel(dst_pad, ones16, z16)
    hs1, dinv16 = _tc_first(xpad, W1, degP)

    agg1 = _agg_kernel(16, hs1, src_pad, dst_pad, z16)
    hs2 = _tc_mid(16, 32, agg1, hs1, dinv16, b1.reshape(1, 16), W2)

    agg2 = _agg_kernel(32, hs2, src_pad, dst_pad, z32)
    hs3 = _tc_mid(32, 64, agg2, hs2, dinv16, b2.reshape(1, 32), W3)

    agg3 = _agg_kernel(64, hs3, src_pad, dst_pad, z64)
    out = _tc_final(agg3, hs3, dinv16, b3.reshape(1, 64),
                    M1, mb1.reshape(1, 32), M2, mb2.reshape(1, 16),
                    M3, mb3.reshape(1, 1))
    return out[:N]


# speedup vs baseline: 21.1086x; 1.2486x over previous
import functools

import jax
import jax.numpy as jnp
from jax import lax
from jax.experimental import pallas as pl
from jax.experimental.pallas import tpu as pltpu
from jax.experimental.pallas import tpu_sc as plsc

N = 10000
E = 320000
NPAD = 10240
NC = 2
NS = 16
NW = NC * NS
CHUNK = 128
EPT = 10240
E_PAD = EPT * NW
NB = 8
NCH = EPT // CHUNK
NPH = NCH // NB
ROWS_PER_TILE = NPAD // NS
BLK = 1024
GRID = NPAD // BLK


def _sc_mesh():
    return plsc.VectorSubcoreMesh(core_axis_name="c", subcore_axis_name="s")


_SC_PARAMS = pltpu.CompilerParams(use_tc_tiling_on_sc=False)


def _deg_kernel(dstb, ones_hbm, zeros_hbm):
    @functools.partial(
        pl.kernel,
        out_type=jax.ShapeDtypeStruct((NC, NPAD, 16), jnp.float32),
        mesh=_sc_mesh(),
        scratch_types=[
            pltpu.VMEM((CHUNK, 16), jnp.float32),
            pltpu.VMEM((NCH, CHUNK), jnp.int32),
            pltpu.VMEM_SHARED((NPAD, 16), jnp.float32),
            tuple(pltpu.SemaphoreType.DMA for _ in range(NB)),
        ],
        compiler_params=_SC_PARAMS,
    )
    def k(dst_hbm, ones_h, zeros_h, out_hbm, ones_v, didx, acc, sems):
        c = lax.axis_index("c")
        s = lax.axis_index("s")
        wid = c * NS + s
        pltpu.sync_copy(ones_h, ones_v)
        pltpu.sync_copy(dst_hbm.at[wid], didx)
        pltpu.sync_copy(zeros_h, acc.at[pl.ds(s * ROWS_PER_TILE, ROWS_PER_TILE)])
        plsc.subcore_barrier()

        for b in range(NB):
            pltpu.async_copy(ones_v, acc.at[didx.at[b]], sems[b], add=True)

        def phase(it, _):
            base = it * NB
            for b in range(NB):
                pltpu.make_async_copy(ones_v, acc.at[didx.at[0]],
                                      sems[b]).wait()
                pltpu.async_copy(ones_v, acc.at[didx.at[base + b]],
                                 sems[b], add=True)
            return 0

        lax.fori_loop(1, NPH, phase, 0)
        for b in range(NB):
            pltpu.make_async_copy(ones_v, acc.at[didx.at[0]], sems[b]).wait()
        plsc.subcore_barrier()
        pltpu.sync_copy(acc.at[pl.ds(s * ROWS_PER_TILE, ROWS_PER_TILE)],
                        out_hbm.at[c, pl.ds(s * ROWS_PER_TILE, ROWS_PER_TILE)])

    return k(dstb, ones_hbm, zeros_hbm)


def _agg_kernel(F, hs, srcb, dstb, zeros_hbm):
    @functools.partial(
        pl.kernel,
        out_type=jax.ShapeDtypeStruct((NC, NPAD, F), jnp.float32),
        mesh=_sc_mesh(),
        scratch_types=[
            pltpu.VMEM((NCH, CHUNK), jnp.int32),
            pltpu.VMEM((NCH, CHUNK), jnp.int32),
            pltpu.VMEM((NB, CHUNK, F), jnp.float32),
            pltpu.VMEM_SHARED((NPAD, F), jnp.float32),
            tuple(pltpu.SemaphoreType.DMA for _ in range(NB)),
            tuple(pltpu.SemaphoreType.DMA for _ in range(NB)),
        ],
        compiler_params=_SC_PARAMS,
    )
    def k(hs_hbm, src_hbm, dst_hbm, zeros_h, out_hbm,
          sidx, didx, rows, acc, gsems, ssems):
        c = lax.axis_index("c")
        s = lax.axis_index("s")
        wid = c * NS + s
        pltpu.sync_copy(src_hbm.at[wid], sidx)
        pltpu.sync_copy(dst_hbm.at[wid], didx)
        pltpu.sync_copy(zeros_h, acc.at[pl.ds(s * ROWS_PER_TILE, ROWS_PER_TILE)])
        plsc.subcore_barrier()

        for b in range(NB):
            pltpu.async_copy(hs_hbm.at[sidx.at[b]], rows.at[b], gsems[b])

        def phase(it, prefetch):
            base = it * NB

            def run(b, nxt):
                pltpu.make_async_copy(hs_hbm.at[sidx.at[b]], rows.at[b],
                                      gsems[b]).wait()
                pltpu.async_copy(rows.at[b], acc.at[didx.at[base + b]],
                                 ssems[b], add=True)
                if nxt is not None:
                    pltpu.make_async_copy(rows.at[b], acc.at[didx.at[0]],
                                          ssems[b]).wait()
                    pltpu.async_copy(hs_hbm.at[sidx.at[nxt + b]], rows.at[b],
                                     gsems[b])

            for b in range(NB):
                run(b, prefetch(it) if callable(prefetch) else prefetch)

        def steady(it, _):
            phase(it, lambda i: (i + 1) * NB)
            return 0

        lax.fori_loop(0, NPH - 1, steady, 0)
        phase(NPH - 1, None)
        for b in range(NB):
            pltpu.make_async_copy(rows.at[b], acc.at[didx.at[0]],
                                  ssems[b]).wait()
        plsc.subcore_barrier()
        pltpu.sync_copy(acc.at[pl.ds(s * ROWS_PER_TILE, ROWS_PER_TILE)],
                        out_hbm.at[c, pl.ds(s * ROWS_PER_TILE, ROWS_PER_TILE)])

    return k(hs, srcb, dstb, zeros_hbm)


def _tc_first(xpad, W1, degP):
    def body(x_ref, w_ref, d_ref, hs_ref, dinv_ref):
        deg = d_ref[0] + d_ref[1] + 1.0
        dinv = lax.rsqrt(deg)
        dinv_ref[...] = dinv
        h = jnp.dot(x_ref[...], w_ref[...],
                    preferred_element_type=jnp.float32)
        hs_ref[...] = h * dinv

    return pl.pallas_call(
        body,
        grid=(GRID,),
        in_specs=[
            pl.BlockSpec((BLK, 128), lambda i: (i, 0)),
            pl.BlockSpec((128, 16), lambda i: (0, 0)),
            pl.BlockSpec((2, BLK, 16), lambda i: (0, i, 0)),
        ],
        out_specs=[
            pl.BlockSpec((BLK, 16), lambda i: (i, 0)),
            pl.BlockSpec((BLK, 16), lambda i: (i, 0)),
        ],
        out_shape=[
            jax.ShapeDtypeStruct((NPAD, 16), jnp.float32),
            jax.ShapeDtypeStruct((NPAD, 16), jnp.float32),
        ],
    )(xpad, W1, degP)


def _tc_mid(F, Fn, aggP, hs, dinv16, b, Wn):
    def body(a_ref, hs_ref, dinv_ref, b_ref, w_ref, out_ref):
        dinv_f = jnp.concatenate([dinv_ref[...]] * (F // 16), axis=1)
        agg = a_ref[0] + a_ref[1] + hs_ref[...]
        h = jax.nn.relu(agg * dinv_f + b_ref[...])
        hn = jnp.dot(h, w_ref[...], preferred_element_type=jnp.float32)
        dinv_fn = jnp.concatenate([dinv_ref[...]] * (Fn // 16), axis=1)
        out_ref[...] = hn * dinv_fn

    return pl.pallas_call(
        body,
        grid=(GRID,),
        in_specs=[
            pl.BlockSpec((2, BLK, F), lambda i: (0, i, 0)),
            pl.BlockSpec((BLK, F), lambda i: (i, 0)),
            pl.BlockSpec((BLK, 16), lambda i: (i, 0)),
            pl.BlockSpec((1, F), lambda i: (0, 0)),
            pl.BlockSpec((F, Fn), lambda i: (0, 0)),
        ],
        out_specs=pl.BlockSpec((BLK, Fn), lambda i: (i, 0)),
        out_shape=jax.ShapeDtypeStruct((NPAD, Fn), jnp.float32),
    )(aggP, hs, dinv16, b, Wn)


def _tc_final(aggP, hs, dinv16, b3, M1, mb1, M2, mb2, M3, mb3):
    def body(a_ref, hs_ref, dinv_ref, b3_ref, m1_ref, mb1_ref, m2_ref,
             mb2_ref, m3_ref, mb3_ref, out_ref):
        dinv64 = jnp.concatenate([dinv_ref[...]] * 4, axis=1)
        agg = a_ref[0] + a_ref[1] + hs_ref[...]
        h = jax.nn.relu(agg * dinv64 + b3_ref[...])
        h = jax.nn.relu(jnp.dot(h, m1_ref[...],
                                preferred_element_type=jnp.float32)
                        + mb1_ref[...])
        h = jax.nn.relu(jnp.dot(h, m2_ref[...],
                                preferred_element_type=jnp.float32)
                        + mb2_ref[...])
        out_ref[...] = (jnp.dot(h, m3_ref[...],
                                preferred_element_type=jnp.float32)
                        + mb3_ref[...])

    return pl.pallas_call(
        body,
        grid=(GRID,),
        in_specs=[
            pl.BlockSpec((2, BLK, 64), lambda i: (0, i, 0)),
            pl.BlockSpec((BLK, 64), lambda i: (i, 0)),
            pl.BlockSpec((BLK, 16), lambda i: (i, 0)),
            pl.BlockSpec((1, 64), lambda i: (0, 0)),
            pl.BlockSpec((64, 32), lambda i: (0, 0)),
            pl.BlockSpec((1, 32), lambda i: (0, 0)),
            pl.BlockSpec((32, 16), lambda i: (0, 0)),
            pl.BlockSpec((1, 16), lambda i: (0, 0)),
            pl.BlockSpec((16, 1), lambda i: (0, 0)),
            pl.BlockSpec((1, 1), lambda i: (0, 0)),
        ],
        out_specs=pl.BlockSpec((BLK, 1), lambda i: (i, 0)),
        out_shape=jax.ShapeDtypeStruct((NPAD, 1), jnp.float32),
    )(aggP, hs, dinv16, b3, M1, mb1, M2, mb2, M3, mb3)


def kernel(x, edge_index, W1, b1, W2, b2, W3, b3, M1, mb1, M2, mb2, M3, mb3):
    src = edge_index[0]
    dst = edge_index[1]
    src_pad = jnp.concatenate(
        [src, jnp.zeros((E_PAD - E,), jnp.int32)])
    dst_pad = jnp.concatenate(
        [dst, N + jnp.arange(E_PAD - E, dtype=jnp.int32) % (NPAD - N)])
    srcb = src_pad.reshape(NW, NCH, CHUNK)
    dstb = dst_pad.reshape(NW, NCH, CHUNK)
    xpad = jnp.pad(x, ((0, NPAD - N), (0, 0)))

    ones16 = jnp.ones((CHUNK, 16), jnp.float32)
    z16 = jnp.zeros((ROWS_PER_TILE, 16), jnp.float32)
    z32 = jnp.zeros((ROWS_PER_TILE, 32), jnp.float32)
    z64 = jnp.zeros((ROWS_PER_TILE, 64), jnp.float32)

    degP = _deg_kernel(dstb, ones16, z16)
    hs1, dinv16 = _tc_first(xpad, W1, degP)

    agg1 = _agg_kernel(16, hs1, srcb, dstb, z16)
    hs2 = _tc_mid(16, 32, agg1, hs1, dinv16, b1.reshape(1, 16), W2)

    agg2 = _agg_kernel(32, hs2, srcb, dstb, z32)
    hs3 = _tc_mid(32, 64, agg2, hs2, dinv16, b2.reshape(1, 32), W3)

    agg3 = _agg_kernel(64, hs3, srcb, dstb, z64)
    out = _tc_final(agg3, hs3, dinv16, b3.reshape(1, 64),
                    M1, mb1.reshape(1, 32), M2, mb2.reshape(1, 16),
                    M3, mb3.reshape(1, 1))
    return out[:N]


# speedup vs baseline: 45.2355x; 2.1430x over previous
import functools

import jax
import jax.numpy as jnp
from jax import lax
from jax.experimental import pallas as pl
from jax.experimental.pallas import tpu as pltpu
from jax.experimental.pallas import tpu_sc as plsc

N = 10000
E = 320000
NPAD = 10240
NC = 2
NS = 16
NW = NC * NS
CHUNK = 128
EPT = 10240
E_PAD = EPT * NW
NB = 8
NCH = EPT // CHUNK
NPH = NCH // NB
ROWS_PER_TILE = NPAD // NS
BLK = 1024
GRID = NPAD // BLK


def _sc_mesh():
    return plsc.VectorSubcoreMesh(core_axis_name="c", subcore_axis_name="s")


_SC_PARAMS = pltpu.CompilerParams(use_tc_tiling_on_sc=False)


def _deg_kernel(dstb, ones_hbm, zeros_hbm):
    @functools.partial(
        pl.kernel,
        out_type=jax.ShapeDtypeStruct((NC, NPAD, 16), jnp.float32),
        mesh=_sc_mesh(),
        scratch_types=[
            pltpu.VMEM((CHUNK, 16), jnp.float32),
            pltpu.VMEM((NCH, CHUNK), jnp.int32),
            pltpu.VMEM_SHARED((NPAD, 16), jnp.float32),
            tuple(pltpu.SemaphoreType.DMA for _ in range(NB)),
        ],
        compiler_params=_SC_PARAMS,
    )
    def k(dst_hbm, ones_h, zeros_h, out_hbm, ones_v, didx, acc, sems):
        c = lax.axis_index("c")
        s = lax.axis_index("s")
        wid = c * NS + s
        pltpu.sync_copy(ones_h, ones_v)
        pltpu.sync_copy(dst_hbm.at[wid], didx)
        pltpu.sync_copy(zeros_h, acc.at[pl.ds(s * ROWS_PER_TILE, ROWS_PER_TILE)])
        plsc.subcore_barrier()

        for b in range(NB):
            pltpu.async_copy(ones_v, acc.at[didx.at[b]], sems[b], add=True)

        def phase(it, _):
            base = it * NB
            for b in range(NB):
                pltpu.make_async_copy(ones_v, acc.at[didx.at[0]],
                                      sems[b]).wait()
                pltpu.async_copy(ones_v, acc.at[didx.at[base + b]],
                                 sems[b], add=True)
            return 0

        lax.fori_loop(1, NPH, phase, 0)
        for b in range(NB):
            pltpu.make_async_copy(ones_v, acc.at[didx.at[0]], sems[b]).wait()
        plsc.subcore_barrier()
        pltpu.sync_copy(acc.at[pl.ds(s * ROWS_PER_TILE, ROWS_PER_TILE)],
                        out_hbm.at[c, pl.ds(s * ROWS_PER_TILE, ROWS_PER_TILE)])

    return k(dstb, ones_hbm, zeros_hbm)


def _agg_kernel(F, hs, srcb, dstb, zeros_hbm):
    @functools.partial(
        pl.kernel,
        out_type=jax.ShapeDtypeStruct((NC, NPAD, F), jnp.float32),
        mesh=_sc_mesh(),
        scratch_types=[
            pltpu.VMEM((NCH, CHUNK), jnp.int32),
            pltpu.VMEM((NCH, CHUNK), jnp.int32),
            pltpu.VMEM((NB, CHUNK, F), jnp.float32),
            pltpu.VMEM_SHARED((NPAD, F), jnp.float32),
            tuple(pltpu.SemaphoreType.DMA for _ in range(NB)),
            tuple(pltpu.SemaphoreType.DMA for _ in range(NB)),
        ],
        compiler_params=_SC_PARAMS,
    )
    def k(hs_hbm, src_hbm, dst_hbm, zeros_h, out_hbm,
          sidx, didx, rows, acc, gsems, ssems):
        c = lax.axis_index("c")
        s = lax.axis_index("s")
        wid = c * NS + s
        pltpu.sync_copy(src_hbm.at[wid], sidx)
        pltpu.sync_copy(dst_hbm.at[wid], didx)
        pltpu.sync_copy(zeros_h, acc.at[pl.ds(s * ROWS_PER_TILE, ROWS_PER_TILE)])
        plsc.subcore_barrier()

        for b in range(NB):
            pltpu.async_copy(hs_hbm.at[sidx.at[b]], rows.at[b], gsems[b])

        def phase(it, prefetch):
            base = it * NB

            def run(b, nxt):
                pltpu.make_async_copy(hs_hbm.at[sidx.at[b]], rows.at[b],
                                      gsems[b]).wait()
                pltpu.async_copy(rows.at[b], acc.at[didx.at[base + b]],
                                 ssems[b], add=True)
                if nxt is not None:
                    pltpu.make_async_copy(rows.at[b], acc.at[didx.at[0]],
                                          ssems[b]).wait()
                    pltpu.async_copy(hs_hbm.at[sidx.at[nxt + b]], rows.at[b],
                                     gsems[b])

            for b in range(NB):
                run(b, prefetch(it) if callable(prefetch) else prefetch)

        def steady(it, _):
            phase(it, lambda i: (i + 1) * NB)
            return 0

        lax.fori_loop(0, NPH - 1, steady, 0)
        phase(NPH - 1, None)
        for b in range(NB):
            pltpu.make_async_copy(rows.at[b], acc.at[didx.at[0]],
                                  ssems[b]).wait()
        plsc.subcore_barrier()
        pltpu.sync_copy(acc.at[pl.ds(s * ROWS_PER_TILE, ROWS_PER_TILE)],
                        out_hbm.at[c, pl.ds(s * ROWS_PER_TILE, ROWS_PER_TILE)])

    return k(hs, srcb, dstb, zeros_hbm)


def _tc_first(xpad, W1, degP):
    def body(x_ref, w_ref, d_ref, hs_ref, dinv_ref):
        deg = d_ref[0] + d_ref[1] + 1.0
        dinv = lax.rsqrt(deg)
        dinv_ref[...] = dinv
        h = jnp.dot(x_ref[...], w_ref[...],
                    preferred_element_type=jnp.float32)
        hs_ref[...] = h * dinv

    return pl.pallas_call(
        body,
        grid=(GRID,),
        in_specs=[
            pl.BlockSpec((BLK, 128), lambda i: (i, 0)),
            pl.BlockSpec((128, 16), lambda i: (0, 0)),
            pl.BlockSpec((2, BLK, 16), lambda i: (0, i, 0)),
        ],
        out_specs=[
            pl.BlockSpec((BLK, 16), lambda i: (i, 0)),
            pl.BlockSpec((BLK, 16), lambda i: (i, 0)),
        ],
        out_shape=[
            jax.ShapeDtypeStruct((NPAD, 16), jnp.float32),
            jax.ShapeDtypeStruct((NPAD, 16), jnp.float32),
        ],
    )(xpad, W1, degP)


def _tc_mid(F, Fn, aggP, hs, dinv16, b, Wn):
    def body(a_ref, hs_ref, dinv_ref, b_ref, w_ref, out_ref):
        dinv_f = jnp.concatenate([dinv_ref[...]] * (F // 16), axis=1)
        agg = a_ref[0] + a_ref[1] + hs_ref[...]
        h = jax.nn.relu(agg * dinv_f + b_ref[...])
        hn = jnp.dot(h, w_ref[...], preferred_element_type=jnp.float32)
        dinv_fn = jnp.concatenate([dinv_ref[...]] * (Fn // 16), axis=1)
        out_ref[...] = hn * dinv_fn

    return pl.pallas_call(
        body,
        grid=(GRID,),
        in_specs=[
            pl.BlockSpec((2, BLK, F), lambda i: (0, i, 0)),
            pl.BlockSpec((BLK, F), lambda i: (i, 0)),
            pl.BlockSpec((BLK, 16), lambda i: (i, 0)),
            pl.BlockSpec((1, F), lambda i: (0, 0)),
            pl.BlockSpec((F, Fn), lambda i: (0, 0)),
        ],
        out_specs=pl.BlockSpec((BLK, Fn), lambda i: (i, 0)),
        out_shape=jax.ShapeDtypeStruct((NPAD, Fn), jnp.float32),
    )(aggP, hs, dinv16, b, Wn)


def _tc_final(aggP, hs, dinv16, b3, M1, mb1, M2, mb2, M3, mb3):
    def body(a_ref, hs_ref, dinv_ref, b3_ref, m1_ref, mb1_ref, m2_ref,
             mb2_ref, m3_ref, mb3_ref, out_ref):
        dinv64 = jnp.concatenate([dinv_ref[...]] * 4, axis=1)
        agg = a_ref[0] + a_ref[1] + hs_ref[...]
        h = jax.nn.relu(agg * dinv64 + b3_ref[...])
        h = jax.nn.relu(jnp.dot(h, m1_ref[...],
                                preferred_element_type=jnp.float32)
                        + mb1_ref[...])
        h = jax.nn.relu(jnp.dot(h, m2_ref[...],
                                preferred_element_type=jnp.float32)
                        + mb2_ref[...])
        out_ref[...] = (jnp.dot(h, m3_ref[...],
                                preferred_element_type=jnp.float32)
                        + mb3_ref[...])

    return pl.pallas_call(
        body,
        grid=(GRID,),
        in_specs=[
            pl.BlockSpec((2, BLK, 64), lambda i: (0, i, 0)),
            pl.BlockSpec((BLK, 64), lambda i: (i, 0)),
            pl.BlockSpec((BLK, 16), lambda i: (i, 0)),
            pl.BlockSpec((1, 64), lambda i: (0, 0)),
            pl.BlockSpec((64, 32), lambda i: (0, 0)),
            pl.BlockSpec((1, 32), lambda i: (0, 0)),
            pl.BlockSpec((32, 16), lambda i: (0, 0)),
            pl.BlockSpec((1, 16), lambda i: (0, 0)),
            pl.BlockSpec((16, 1), lambda i: (0, 0)),
            pl.BlockSpec((1, 1), lambda i: (0, 0)),
        ],
        out_specs=pl.BlockSpec((BLK, 1), lambda i: (i, 0)),
        out_shape=jax.ShapeDtypeStruct((NPAD, 1), jnp.float32),
    )(aggP, hs, dinv16, b3, M1, mb1, M2, mb2, M3, mb3)


def kernel(x, edge_index, W1, b1, W2, b2, W3, b3, M1, mb1, M2, mb2, M3, mb3):
    src = edge_index[0]
    dst = edge_index[1]
    src_pad = jnp.concatenate(
        [src, jnp.arange(E_PAD - E, dtype=jnp.int32) * 13 % N])
    dst_pad = jnp.concatenate(
        [dst, N + jnp.arange(E_PAD - E, dtype=jnp.int32) % (NPAD - N)])
    srcb = src_pad.reshape(NW, NCH, CHUNK)
    dstb = dst_pad.reshape(NW, NCH, CHUNK)
    xpad = jnp.pad(x, ((0, NPAD - N), (0, 0)))

    ones16 = jnp.ones((CHUNK, 16), jnp.float32)
    z16 = jnp.zeros((ROWS_PER_TILE, 16), jnp.float32)
    z32 = jnp.zeros((ROWS_PER_TILE, 32), jnp.float32)
    z64 = jnp.zeros((ROWS_PER_TILE, 64), jnp.float32)

    degP = _deg_kernel(dstb, ones16, z16)
    hs1, dinv16 = _tc_first(xpad, W1, degP)

    agg1 = _agg_kernel(16, hs1, srcb, dstb, z16)
    hs2 = _tc_mid(16, 32, agg1, hs1, dinv16, b1.reshape(1, 16), W2)

    agg2 = _agg_kernel(32, hs2, srcb, dstb, z32)
    hs3 = _tc_mid(32, 64, agg2, hs2, dinv16, b2.reshape(1, 32), W3)

    agg3 = _agg_kernel(64, hs3, srcb, dstb, z64)
    out = _tc_final(agg3, hs3, dinv16, b3.reshape(1, 64),
                    M1, mb1.reshape(1, 32), M2, mb2.reshape(1, 16),
                    M3, mb3.reshape(1, 1))
    return out[:N]


# speedup vs baseline: 46.5633x; 1.0294x over previous
import functools

import jax
import jax.numpy as jnp
from jax import lax
from jax.experimental import pallas as pl
from jax.experimental.pallas import tpu as pltpu
from jax.experimental.pallas import tpu_sc as plsc

N = 10000
E = 320000
NPAD = 10240
NC = 2
NS = 16
NW = NC * NS
CHUNK = 128
EPT = 10240
E_PAD = EPT * NW
NB = 8
NCH = EPT // CHUNK
NPH = NCH // NB
ROWS_PER_TILE = NPAD // NS
BLK = 2560
GRID = NPAD // BLK


def _sc_mesh():
    return plsc.VectorSubcoreMesh(core_axis_name="c", subcore_axis_name="s")


_SC_PARAMS = pltpu.CompilerParams(use_tc_tiling_on_sc=False)


def _deg_kernel(dstb, ones_hbm, zeros_hbm):
    @functools.partial(
        pl.kernel,
        out_type=jax.ShapeDtypeStruct((NC, NPAD, 16), jnp.float32),
        mesh=_sc_mesh(),
        scratch_types=[
            pltpu.VMEM((CHUNK, 16), jnp.float32),
            pltpu.VMEM((NCH, CHUNK), jnp.int32),
            pltpu.VMEM_SHARED((NPAD, 16), jnp.float32),
            tuple(pltpu.SemaphoreType.DMA for _ in range(NB)),
        ],
        compiler_params=_SC_PARAMS,
    )
    def k(dst_hbm, ones_h, zeros_h, out_hbm, ones_v, didx, acc, sems):
        c = lax.axis_index("c")
        s = lax.axis_index("s")
        wid = c * NS + s
        pltpu.sync_copy(ones_h, ones_v)
        pltpu.sync_copy(dst_hbm.at[wid], didx)
        pltpu.sync_copy(zeros_h, acc.at[pl.ds(s * ROWS_PER_TILE, ROWS_PER_TILE)])
        plsc.subcore_barrier()

        for b in range(NB):
            pltpu.async_copy(ones_v, acc.at[didx.at[b]], sems[b], add=True)

        def phase(it, _):
            base = it * NB
            for b in range(NB):
                pltpu.make_async_copy(ones_v, acc.at[didx.at[0]],
                                      sems[b]).wait()
                pltpu.async_copy(ones_v, acc.at[didx.at[base + b]],
                                 sems[b], add=True)
            return 0

        lax.fori_loop(1, NPH, phase, 0)
        for b in range(NB):
            pltpu.make_async_copy(ones_v, acc.at[didx.at[0]], sems[b]).wait()
        plsc.subcore_barrier()
        pltpu.sync_copy(acc.at[pl.ds(s * ROWS_PER_TILE, ROWS_PER_TILE)],
                        out_hbm.at[c, pl.ds(s * ROWS_PER_TILE, ROWS_PER_TILE)])

    return k(dstb, ones_hbm, zeros_hbm)


def _agg_kernel(F, hs, srcb, dstb, zeros_hbm):
    @functools.partial(
        pl.kernel,
        out_type=jax.ShapeDtypeStruct((NC, NPAD, F), jnp.float32),
        mesh=_sc_mesh(),
        scratch_types=[
            pltpu.VMEM((NCH, CHUNK), jnp.int32),
            pltpu.VMEM((NCH, CHUNK), jnp.int32),
            pltpu.VMEM((NB, CHUNK, F), jnp.float32),
            pltpu.VMEM_SHARED((NPAD, F), jnp.float32),
            tuple(pltpu.SemaphoreType.DMA for _ in range(NB)),
            tuple(pltpu.SemaphoreType.DMA for _ in range(NB)),
        ],
        compiler_params=_SC_PARAMS,
    )
    def k(hs_hbm, src_hbm, dst_hbm, zeros_h, out_hbm,
          sidx, didx, rows, acc, gsems, ssems):
        c = lax.axis_index("c")
        s = lax.axis_index("s")
        wid = c * NS + s
        pltpu.sync_copy(src_hbm.at[wid], sidx)
        pltpu.sync_copy(dst_hbm.at[wid], didx)
        pltpu.sync_copy(zeros_h, acc.at[pl.ds(s * ROWS_PER_TILE, ROWS_PER_TILE)])
        plsc.subcore_barrier()

        for b in range(NB):
            pltpu.async_copy(hs_hbm.at[sidx.at[b]], rows.at[b], gsems[b])

        def phase(it, prefetch):
            base = it * NB

            def run(b, nxt):
                pltpu.make_async_copy(hs_hbm.at[sidx.at[b]], rows.at[b],
                                      gsems[b]).wait()
                pltpu.async_copy(rows.at[b], acc.at[didx.at[base + b]],
                                 ssems[b], add=True)
                if nxt is not None:
                    pltpu.make_async_copy(rows.at[b], acc.at[didx.at[0]],
                                          ssems[b]).wait()
                    pltpu.async_copy(hs_hbm.at[sidx.at[nxt + b]], rows.at[b],
                                     gsems[b])

            for b in range(NB):
                run(b, prefetch(it) if callable(prefetch) else prefetch)

        def steady(it, _):
            phase(it, lambda i: (i + 1) * NB)
            return 0

        lax.fori_loop(0, NPH - 1, steady, 0)
        phase(NPH - 1, None)
        for b in range(NB):
            pltpu.make_async_copy(rows.at[b], acc.at[didx.at[0]],
                                  ssems[b]).wait()
        plsc.subcore_barrier()
        pltpu.sync_copy(acc.at[pl.ds(s * ROWS_PER_TILE, ROWS_PER_TILE)],
                        out_hbm.at[c, pl.ds(s * ROWS_PER_TILE, ROWS_PER_TILE)])

    return k(hs, srcb, dstb, zeros_hbm)


def _tc_mm1(xpad, W1):
    def body(x_ref, w_ref, out_ref):
        out_ref[...] = jnp.dot(x_ref[...], w_ref[...],
                               preferred_element_type=jnp.float32)

    return pl.pallas_call(
        body,
        grid=(GRID,),
        in_specs=[
            pl.BlockSpec((BLK, 128), lambda i: (i, 0)),
            pl.BlockSpec((128, 16), lambda i: (0, 0)),
        ],
        out_specs=pl.BlockSpec((BLK, 16), lambda i: (i, 0)),
        out_shape=jax.ShapeDtypeStruct((NPAD, 16), jnp.float32),
    )(xpad, W1)


def _tc_scale1(h1p, degP):
    def body(h_ref, d_ref, hs_ref, dinv_ref):
        deg = d_ref[0] + d_ref[1] + 1.0
        dinv = lax.rsqrt(deg)
        dinv_ref[...] = dinv
        hs_ref[...] = h_ref[...] * dinv

    return pl.pallas_call(
        body,
        grid=(GRID,),
        in_specs=[
            pl.BlockSpec((BLK, 16), lambda i: (i, 0)),
            pl.BlockSpec((2, BLK, 16), lambda i: (0, i, 0)),
        ],
        out_specs=[
            pl.BlockSpec((BLK, 16), lambda i: (i, 0)),
            pl.BlockSpec((BLK, 16), lambda i: (i, 0)),
        ],
        out_shape=[
            jax.ShapeDtypeStruct((NPAD, 16), jnp.float32),
            jax.ShapeDtypeStruct((NPAD, 16), jnp.float32),
        ],
    )(h1p, degP)


def _tc_mid(F, Fn, aggP, hs, dinv16, b, Wn):
    def body(a_ref, hs_ref, dinv_ref, b_ref, w_ref, out_ref):
        dinv_f = jnp.concatenate([dinv_ref[...]] * (F // 16), axis=1)
        agg = a_ref[0] + a_ref[1] + hs_ref[...]
        h = jax.nn.relu(agg * dinv_f + b_ref[...])
        hn = jnp.dot(h, w_ref[...], preferred_element_type=jnp.float32)
        dinv_fn = jnp.concatenate([dinv_ref[...]] * (Fn // 16), axis=1)
        out_ref[...] = hn * dinv_fn

    return pl.pallas_call(
        body,
        grid=(GRID,),
        in_specs=[
            pl.BlockSpec((2, BLK, F), lambda i: (0, i, 0)),
            pl.BlockSpec((BLK, F), lambda i: (i, 0)),
            pl.BlockSpec((BLK, 16), lambda i: (i, 0)),
            pl.BlockSpec((1, F), lambda i: (0, 0)),
            pl.BlockSpec((F, Fn), lambda i: (0, 0)),
        ],
        out_specs=pl.BlockSpec((BLK, Fn), lambda i: (i, 0)),
        out_shape=jax.ShapeDtypeStruct((NPAD, Fn), jnp.float32),
    )(aggP, hs, dinv16, b, Wn)


def _tc_final(aggP, hs, dinv16, b3, M1, mb1, M2, mb2, M3, mb3):
    def body(a_ref, hs_ref, dinv_ref, b3_ref, m1_ref, mb1_ref, m2_ref,
             mb2_ref, m3_ref, mb3_ref, out_ref):
        dinv64 = jnp.concatenate([dinv_ref[...]] * 4, axis=1)
        agg = a_ref[0] + a_ref[1] + hs_ref[...]
        h = jax.nn.relu(agg * dinv64 + b3_ref[...])
        h = jax.nn.relu(jnp.dot(h, m1_ref[...],
                                preferred_element_type=jnp.float32)
                        + mb1_ref[...])
        h = jax.nn.relu(jnp.dot(h, m2_ref[...],
                                preferred_element_type=jnp.float32)
                        + mb2_ref[...])
        out_ref[...] = (jnp.dot(h, m3_ref[...],
                                preferred_element_type=jnp.float32)
                        + mb3_ref[...])

    return pl.pallas_call(
        body,
        grid=(GRID,),
        in_specs=[
            pl.BlockSpec((2, BLK, 64), lambda i: (0, i, 0)),
            pl.BlockSpec((BLK, 64), lambda i: (i, 0)),
            pl.BlockSpec((BLK, 16), lambda i: (i, 0)),
            pl.BlockSpec((1, 64), lambda i: (0, 0)),
            pl.BlockSpec((64, 32), lambda i: (0, 0)),
            pl.BlockSpec((1, 32), lambda i: (0, 0)),
            pl.BlockSpec((32, 16), lambda i: (0, 0)),
            pl.BlockSpec((1, 16), lambda i: (0, 0)),
            pl.BlockSpec((16, 1), lambda i: (0, 0)),
            pl.BlockSpec((1, 1), lambda i: (0, 0)),
        ],
        out_specs=pl.BlockSpec((BLK, 1), lambda i: (i, 0)),
        out_shape=jax.ShapeDtypeStruct((NPAD, 1), jnp.float32),
    )(aggP, hs, dinv16, b3, M1, mb1, M2, mb2, M3, mb3)


def kernel(x, edge_index, W1, b1, W2, b2, W3, b3, M1, mb1, M2, mb2, M3, mb3):
    src = edge_index[0]
    dst = edge_index[1]
    src_pad = jnp.concatenate(
        [src, jnp.arange(E_PAD - E, dtype=jnp.int32) * 13 % N])
    dst_pad = jnp.concatenate(
        [dst, N + jnp.arange(E_PAD - E, dtype=jnp.int32) % (NPAD - N)])
    srcb = src_pad.reshape(NW, NCH, CHUNK)
    dstb = dst_pad.reshape(NW, NCH, CHUNK)
    xpad = jnp.pad(x, ((0, NPAD - N), (0, 0)))

    ones16 = jnp.ones((CHUNK, 16), jnp.float32)
    z16 = jnp.zeros((ROWS_PER_TILE, 16), jnp.float32)
    z32 = jnp.zeros((ROWS_PER_TILE, 32), jnp.float32)
    z64 = jnp.zeros((ROWS_PER_TILE, 64), jnp.float32)

    h1p = _tc_mm1(xpad, W1)
    degP = _deg_kernel(dstb, ones16, z16)
    hs1, dinv16 = _tc_scale1(h1p, degP)

    agg1 = _agg_kernel(16, hs1, srcb, dstb, z16)
    hs2 = _tc_mid(16, 32, agg1, hs1, dinv16, b1.reshape(1, 16), W2)

    agg2 = _agg_kernel(32, hs2, srcb, dstb, z32)
    hs3 = _tc_mid(32, 64, agg2, hs2, dinv16, b2.reshape(1, 32), W3)

    agg3 = _agg_kernel(64, hs3, srcb, dstb, z64)
    out = _tc_final(agg3, hs3, dinv16, b3.reshape(1, 64),
                    M1, mb1.reshape(1, 32), M2, mb2.reshape(1, 16),
                    M3, mb3.reshape(1, 1))
    return out[:N]


# speedup vs baseline: 61.2495x; 1.3154x over previous
import functools

import numpy as np

import jax
import jax.numpy as jnp
from jax import lax
from jax.experimental import pallas as pl
from jax.experimental.pallas import tpu as pltpu
from jax.experimental.pallas import tpu_sc as plsc

N = 10000
E = 320000
NPAD = 10240
NC = 2
NS = 16
NW = NC * NS
CHUNK = 128
EPT = 10240
E_PAD = EPT * NW
NB = 8
NCH = EPT // CHUNK
NPH = NCH // NB
ROWS_PER_TILE = NPAD // NS
GRID = 5


def _sc_mesh():
    return plsc.VectorSubcoreMesh(core_axis_name="c", subcore_axis_name="s")


_SC_PARAMS = pltpu.CompilerParams(use_tc_tiling_on_sc=False)

def _widen_sel(f_in):
    sel = np.zeros((128, 256), np.float32)
    k = 128 // f_in
    for i in range(k):
        sel[f_in * i, 2 * f_in * i:2 * f_in * (i + 1)] = 1.0
    return jnp.asarray(sel)


def _deg_kernel(dstb, ones_hbm, zeros_hbm):
    @functools.partial(
        pl.kernel,
        out_type=jax.ShapeDtypeStruct((NC, NPAD, 16), jnp.float32),
        mesh=_sc_mesh(),
        scratch_types=[
            pltpu.VMEM((CHUNK, 16), jnp.float32),
            pltpu.VMEM((NCH, CHUNK), jnp.int32),
            pltpu.VMEM_SHARED((NPAD, 16), jnp.float32),
            tuple(pltpu.SemaphoreType.DMA for _ in range(NB)),
        ],
        compiler_params=_SC_PARAMS,
    )
    def k(dst_hbm, ones_h, zeros_h, out_hbm, ones_v, didx, acc, sems):
        c = lax.axis_index("c")
        s = lax.axis_index("s")
        wid = c * NS + s
        pltpu.sync_copy(ones_h, ones_v)
        pltpu.sync_copy(dst_hbm.at[wid], didx)
        pltpu.sync_copy(zeros_h, acc.at[pl.ds(s * ROWS_PER_TILE, ROWS_PER_TILE)])
        plsc.subcore_barrier()

        for b in range(NB):
            pltpu.async_copy(ones_v, acc.at[didx.at[b]], sems[b], add=True)

        def phase(it, _):
            base = it * NB
            for b in range(NB):
                pltpu.make_async_copy(ones_v, acc.at[didx.at[0]],
                                      sems[b]).wait()
                pltpu.async_copy(ones_v, acc.at[didx.at[base + b]],
                                 sems[b], add=True)
            return 0

        lax.fori_loop(1, NPH, phase, 0)
        for b in range(NB):
            pltpu.make_async_copy(ones_v, acc.at[didx.at[0]], sems[b]).wait()
        plsc.subcore_barrier()
        pltpu.sync_copy(acc.at[pl.ds(s * ROWS_PER_TILE, ROWS_PER_TILE)],
                        out_hbm.at[c, pl.ds(s * ROWS_PER_TILE, ROWS_PER_TILE)])

    return k(dstb, ones_hbm, zeros_hbm)


def _agg_kernel(F, hs, srcb, dstb, zeros_hbm):
    @functools.partial(
        pl.kernel,
        out_type=jax.ShapeDtypeStruct((NC, NPAD, F), jnp.float32),
        mesh=_sc_mesh(),
        scratch_types=[
            pltpu.VMEM((NCH, CHUNK), jnp.int32),
            pltpu.VMEM((NCH, CHUNK), jnp.int32),
            pltpu.VMEM((NB, CHUNK, F), jnp.float32),
            pltpu.VMEM_SHARED((NPAD, F), jnp.float32),
            tuple(pltpu.SemaphoreType.DMA for _ in range(NB)),
            tuple(pltpu.SemaphoreType.DMA for _ in range(NB)),
        ],
        compiler_params=_SC_PARAMS,
    )
    def k(hs_hbm, src_hbm, dst_hbm, zeros_h, out_hbm,
          sidx, didx, rows, acc, gsems, ssems):
        c = lax.axis_index("c")
        s = lax.axis_index("s")
        wid = c * NS + s
        pltpu.sync_copy(src_hbm.at[wid], sidx)
        pltpu.sync_copy(dst_hbm.at[wid], didx)
        pltpu.sync_copy(zeros_h, acc.at[pl.ds(s * ROWS_PER_TILE, ROWS_PER_TILE)])
        plsc.subcore_barrier()

        for b in range(NB):
            pltpu.async_copy(hs_hbm.at[sidx.at[b]], rows.at[b], gsems[b])

        def phase(it, prefetch):
            base = it * NB

            def run(b, nxt):
                pltpu.make_async_copy(hs_hbm.at[sidx.at[b]], rows.at[b],
                                      gsems[b]).wait()
                pltpu.async_copy(rows.at[b], acc.at[didx.at[base + b]],
                                 ssems[b], add=True)
                if nxt is not None:
                    pltpu.make_async_copy(rows.at[b], acc.at[didx.at[0]],
                                          ssems[b]).wait()
                    pltpu.async_copy(hs_hbm.at[sidx.at[nxt + b]], rows.at[b],
                                     gsems[b])

            for b in range(NB):
                run(b, prefetch(it) if callable(prefetch) else prefetch)

        def steady(it, _):
            phase(it, lambda i: (i + 1) * NB)
            return 0

        lax.fori_loop(0, NPH - 1, steady, 0)
        phase(NPH - 1, None)
        for b in range(NB):
            pltpu.make_async_copy(rows.at[b], acc.at[didx.at[0]],
                                  ssems[b]).wait()
        plsc.subcore_barrier()
        pltpu.sync_copy(acc.at[pl.ds(s * ROWS_PER_TILE, ROWS_PER_TILE)],
                        out_hbm.at[c, pl.ds(s * ROWS_PER_TILE, ROWS_PER_TILE)])

    return k(hs, srcb, dstb, zeros_hbm)


def _tc_mm1(x3, W1):
    nr = N // 8

    def body(x_ref, w_ref, out_ref):
        parts = [
            jnp.dot(x_ref[:, k, :], w_ref[...],
                    preferred_element_type=jnp.float32)
            for k in range(8)
        ]
        out_ref[...] = jnp.concatenate(parts, axis=1)

    return pl.pallas_call(
        body,
        out_shape=jax.ShapeDtypeStruct((nr, 128), jnp.float32),
    )(x3, W1)


def _tc_scale1(h1p, degPp):
    nr = N // 8

    def body(h_ref, d_ref, hs_ref, dinv_ref):
        deg = d_ref[0, :nr] + d_ref[1, :nr] + 1.0
        dinv = lax.rsqrt(deg)
        dinv_ref[...] = dinv
        hs_ref[...] = h_ref[...] * dinv

    return pl.pallas_call(
        body,
        out_shape=[
            jax.ShapeDtypeStruct((nr, 128), jnp.float32),
            jax.ShapeDtypeStruct((nr, 128), jnp.float32),
        ],
    )(h1p, degPp)


def _tc_mid(F, aggP, hs, dinv, bt, Wbd, sel):
    nr = N * F // 128
    npadr = NPAD * F // 128

    def body(a_ref, hs_ref, dinv_ref, b_ref, w_ref, s_ref, hsn_ref, dn_ref):
        dinv = dinv_ref[...]
        z = a_ref[0, :nr] + a_ref[1, :nr] + hs_ref[...]
        lhs = jax.nn.relu(dinv * dinv * z + dinv * b_ref[...])
        hsn_ref[...] = jnp.dot(lhs, w_ref[...],
                               preferred_element_type=jnp.float32)
        dn_ref[...] = jnp.dot(dinv, s_ref[...],
                              preferred_element_type=jnp.float32)

    hsn, dinvn = pl.pallas_call(
        body,
        out_shape=[
            jax.ShapeDtypeStruct((nr, 256), jnp.float32),
            jax.ShapeDtypeStruct((nr, 256), jnp.float32),
        ],
    )(aggP.reshape(NC, npadr, 128), hs, dinv, bt, Wbd, sel)
    return hsn.reshape(nr * 2, 128), dinvn.reshape(nr * 2, 128)


def _tc_final(aggP, hs, dinv, b3t, M1bd, mb1t, M2bd, mb2t, M3bd, mb3t):
    nr = N * 64 // 128
    blk = nr // GRID
    npadr = NPAD * 64 // 128

    def body(a_ref, hs_ref, dinv_ref, b3_ref, m1_ref, mb1_ref, m2_ref,
             mb2_ref, m3_ref, mb3_ref, out_ref):
        z = a_ref[0] + a_ref[1] + hs_ref[...]
        h = jax.nn.relu(dinv_ref[...] * z + b3_ref[...])
        h = jax.nn.relu(jnp.dot(h, m1_ref[...],
                                preferred_element_type=jnp.float32)
                        + mb1_ref[...])
        h = jax.nn.relu(jnp.dot(h, m2_ref[...],
                                preferred_element_type=jnp.float32)
                        + mb2_ref[...])
        out_ref[...] = (jnp.dot(h, m3_ref[...],
                                preferred_element_type=jnp.float32)
                        + mb3_ref[...])

    return pl.pallas_call(
        body,
        grid=(GRID,),
        in_specs=[
            pl.BlockSpec((2, blk, 128), lambda i: (0, i, 0)),
            pl.BlockSpec((blk, 128), lambda i: (i, 0)),
            pl.BlockSpec((blk, 128), lambda i: (i, 0)),
            pl.BlockSpec((1, 128), lambda i: (0, 0)),
            pl.BlockSpec((128, 64), lambda i: (0, 0)),
            pl.BlockSpec((1, 64), lambda i: (0, 0)),
            pl.BlockSpec((64, 32), lambda i: (0, 0)),
            pl.BlockSpec((1, 32), lambda i: (0, 0)),
            pl.BlockSpec((32, 2), lambda i: (0, 0)),
            pl.BlockSpec((1, 2), lambda i: (0, 0)),
        ],
        out_specs=pl.BlockSpec((blk, 2), lambda i: (i, 0)),
        out_shape=jax.ShapeDtypeStruct((nr, 2), jnp.float32),
    )(aggP.reshape(NC, npadr, 128), hs, dinv, b3t, M1bd, mb1t, M2bd,
      mb2t, M3bd, mb3t)


def kernel(x, edge_index, W1, b1, W2, b2, W3, b3, M1, mb1, M2, mb2, M3, mb3):
    src = edge_index[0]
    dst = edge_index[1]
    src_pad = jnp.concatenate(
        [src, jnp.arange(E_PAD - E, dtype=jnp.int32) * 13 % N])
    dst_pad = jnp.concatenate(
        [dst, N + jnp.arange(E_PAD - E, dtype=jnp.int32) % (NPAD - N)])
    srcb = src_pad.reshape(NW, NCH, CHUNK)
    dstb = dst_pad.reshape(NW, NCH, CHUNK)

    ones16 = jnp.ones((CHUNK, 16), jnp.float32)
    z16 = jnp.zeros((ROWS_PER_TILE, 16), jnp.float32)
    z32 = jnp.zeros((ROWS_PER_TILE, 32), jnp.float32)
    z64 = jnp.zeros((ROWS_PER_TILE, 64), jnp.float32)

    eye8 = jnp.asarray(np.eye(8, dtype=np.float32))
    eye4 = jnp.asarray(np.eye(4, dtype=np.float32))
    eye2 = jnp.asarray(np.eye(2, dtype=np.float32))

    h1p = _tc_mm1(x.reshape(N // 8, 8, 128), W1)
    degP = _deg_kernel(dstb, ones16, z16)
    hs1, dinv16 = _tc_scale1(h1p, degP.reshape(NC, NPAD // 8, 128))

    agg1 = _agg_kernel(16, hs1.reshape(N, 16), srcb, dstb, z16)
    hs2, dinv32 = _tc_mid(
        16, agg1, hs1, dinv16, jnp.tile(b1, 8).reshape(1, 128),
        jnp.kron(eye8, W2), _widen_sel(16))

    agg2 = _agg_kernel(32, hs2.reshape(N, 32), srcb, dstb, z32)
    hs3, dinv64 = _tc_mid(
        32, agg2, hs2, dinv32, jnp.tile(b2, 4).reshape(1, 128),
        jnp.kron(eye4, W3), _widen_sel(32))

    agg3 = _agg_kernel(64, hs3.reshape(N, 64), srcb, dstb, z64)
    out = _tc_final(
        agg3, hs3, dinv64, jnp.tile(b3, 2).reshape(1, 128),
        jnp.kron(eye2, M1), jnp.tile(mb1, 2).reshape(1, 64),
        jnp.kron(eye2, M2), jnp.tile(mb2, 2).reshape(1, 32),
        jnp.kron(eye2, M3), jnp.tile(mb3, 2).reshape(1, 2))
    return out.reshape(N, 1)
